# baseline (speedup 1.0000x reference)
import jax
import jax.numpy as jnp
from jax.experimental import pallas as pl

N = 10000
E = 640000
B = 1
H = 64


def _linear(x, W, b):
    return x @ W.T + b


def _bn(x, g, b, eps=1e-5):
    m = jnp.mean(x, axis=0)
    v = jnp.var(x, axis=0)
    return (x - m) / jnp.sqrt(v + eps) * g + b


def _noop_body(x_ref, o_ref):
    o_ref[...] = x_ref[...]


def kernel(w, edge_index, batch, x_v, params):
    p = params
    src, dst = edge_index[0], edge_index[1]

    def gin(x, Wb):
        aggr = jax.ops.segment_sum(x[src], dst, num_segments=N)
        return _linear(aggr, Wb[0], Wb[1])

    x_v2 = _linear(x_v[:, None], *p['fc1'])
    prob = jax.nn.relu(_bn(gin(w[:, None], p['g1']) + x_v2, *p['bn1']))
    x_v3 = _linear(x_v2, *p['fc2'])
    prob = jax.nn.relu(_bn(gin(prob, p['g2']) + x_v3, *p['bn2']))
    x_v4 = _linear(x_v3, *p['fc3'])
    prob = jax.nn.relu(_bn(gin(prob, p['g3']) + x_v4, *p['bn3']))
    x_v5 = _linear(x_v4, *p['fc4'])
    prob = jax.nn.relu(gin(prob, p['g4']) + x_v5)
    counts = jax.ops.segment_sum(jnp.ones((N,), jnp.float32), batch, num_segments=B)
    gf = jax.ops.segment_sum(prob, batch, num_segments=B) / jnp.maximum(counts, 1.0)[:, None]
    wgf = _linear(gf, *p['q2'])
    wp = _linear(prob, *p['q3'])
    gfe = jnp.broadcast_to(wgf[0][None, :], (N, H))
    cat = jnp.concatenate([gfe, wp], axis=1)
    Q = _linear(jax.nn.relu(cat), *p['q1'])

    Q = pl.pallas_call(
        _noop_body, out_shape=jax.ShapeDtypeStruct(Q.shape, Q.dtype))(Q)

    Q_dense = Q[None, :, :]
    Q_mask = jnp.ones((B, N), dtype=bool)
    adj = jnp.zeros((B, N, N), jnp.float32).at[jnp.zeros_like(src), src, dst].add(1.0)
    return (Q_dense, Q_mask, adj)


# speedup vs baseline: 8.5210x; 8.5210x over previous
import functools

import jax
import jax.numpy as jnp
from jax import lax
from jax.experimental import pallas as pl
from jax.experimental.pallas import tpu as pltpu
from jax.experimental.pallas import tpu_sc as plsc

N = 10000
E = 640000
B = 1
H = 64

NC = 2
NS = 16
NW = NC * NS
EW = E // NW
C = 128
NCH = (EW + C - 1) // C
EPAD = NCH * C
ACC1 = NS * 640
ACCR = N + 16

_mesh = functools.partial(
    plsc.VectorSubcoreMesh, core_axis_name="c", subcore_axis_name="s",
    num_cores=NC, num_subcores=NS)

_F32 = jnp.float32
_I32 = jnp.int32


def _wid():
    return lax.axis_index("s") * NC + lax.axis_index("c")


def _seg1_body(src_hbm, dst_hbm, w_hbm, out_hbm, srcv, dstv, vals, zbuf,
               obuf, acc_sh, sem):
    c = lax.axis_index("c")
    s = lax.axis_index("s")
    w = _wid()
    pltpu.sync_copy(src_hbm.at[w], srcv)
    pltpu.sync_copy(dst_hbm.at[w], dstv)
    z = jnp.zeros((16,), _F32)
    for i in range(40):
        zbuf[pl.ds(i * 16, 16)] = z
    pltpu.sync_copy(zbuf, acc_sh.at[pl.ds(s * 640, 640)])
    plsc.subcore_barrier()

    def chunk(j, _):
        pltpu.async_copy(w_hbm.at[srcv.at[j]], vals, sem).wait()
        pltpu.sync_copy(vals, acc_sh.at[dstv.at[j]], add=True)
        return ()

    lax.fori_loop(0, NCH, chunk, ())
    plsc.subcore_barrier()

    @pl.when(s < 10)
    def _():
        pltpu.sync_copy(acc_sh.at[pl.ds(s * 1000, 1000)], obuf)
        pltpu.sync_copy(obuf, out_hbm.at[pl.ds(c * N + s * 1000, 1000)])


def _seg1(srcp, dstp, w):
    k = pl.kernel(
        _seg1_body,
        out_type=jax.ShapeDtypeStruct((NC * N,), _F32),
        mesh=_mesh(),
        scratch_types=[
            pltpu.VMEM((NCH, C), _I32),
            pltpu.VMEM((NCH, C), _I32),
            pltpu.VMEM((C,), _F32),
            pltpu.VMEM((640,), _F32),
            pltpu.VMEM((1000,), _F32),
            pltpu.VMEM_SHARED((ACC1,), _F32),
            pltpu.SemaphoreType.DMA,
        ],
    )
    return k(srcp, dstp, w)


def _segrow_body(src_hbm, dst_hbm, x_hbm, out_hbm, srcv, dstv, rows, zbuf,
                 obuf, acc_sh, sem):
    c = lax.axis_index("c")
    s = lax.axis_index("s")
    w = _wid()
    pltpu.sync_copy(src_hbm.at[w], srcv)
    pltpu.sync_copy(dst_hbm.at[w], dstv)
    z = jnp.zeros((16,), _F32)

    def zrow(i, _):
        for k in range(H // 16):
            zbuf[i, pl.ds(k * 16, 16)] = z
        return ()

    lax.fori_loop(0, C, zrow, ())
    base = s * 632
    for k in range(4):
        pltpu.sync_copy(zbuf, acc_sh.at[pl.ds(base + k * C, C)])

    @pl.when(s < 15)
    def _():
        pltpu.sync_copy(zbuf.at[pl.ds(0, 120)],
                        acc_sh.at[pl.ds(base + 512, 120)])

    @pl.when(s == 15)
    def _():
        pltpu.sync_copy(zbuf.at[pl.ds(0, 24)],
                        acc_sh.at[pl.ds(9480 + 512, 24)])

    plsc.subcore_barrier()

    def chunk(j, _):
        pltpu.async_copy(x_hbm.at[srcv.at[j]], rows, sem).wait()
        pltpu.sync_copy(rows, acc_sh.at[dstv.at[j]], add=True)
        return ()

    lax.fori_loop(0, NCH, chunk, ())
    plsc.subcore_barrier()

    for k in range(4):
        pltpu.sync_copy(acc_sh.at[pl.ds(base + k * C, C)], obuf)
        pltpu.sync_copy(obuf, out_hbm.at[c, pl.ds(base + k * C, C)])

    @pl.when(s < 15)
    def _():
        pltpu.sync_copy(acc_sh.at[pl.ds(base + 512, 120)],
                        obuf.at[pl.ds(0, 120)])
        pltpu.sync_copy(obuf.at[pl.ds(0, 120)],
                        out_hbm.at[c, pl.ds(base + 512, 120)])

    @pl.when(s == 15)
    def _():
        pltpu.sync_copy(acc_sh.at[pl.ds(9992, 8)], obuf.at[pl.ds(0, 8)])
        pltpu.sync_copy(obuf.at[pl.ds(0, 8)],
                        out_hbm.at[c, pl.ds(9992, 8)])


def _segrow(srcp, dstp, x):
    k = pl.kernel(
        _segrow_body,
        out_type=jax.ShapeDtypeStruct((NC, N, H), _F32),
        mesh=_mesh(),
        compiler_params=pltpu.CompilerParams(use_tc_tiling_on_sc=False),
        scratch_types=[
            pltpu.VMEM((NCH, C), _I32),
            pltpu.VMEM((NCH, C), _I32),
            pltpu.VMEM((C, H), _F32),
            pltpu.VMEM((C, H), _F32),
            pltpu.VMEM((C, H), _F32),
            pltpu.VMEM_SHARED((ACCR, H), _F32),
            pltpu.SemaphoreType.DMA,
        ],
    )
    return k(srcp, dstp, x)


def _dot(a, b):
    return lax.dot_general(a, b, (((1,), (0,)), ((), ())),
                           precision=lax.Precision.HIGHEST,
                           preferred_element_type=_F32)


def _bn_relu(t, g, b):
    m = jnp.mean(t, axis=0, keepdims=True)
    v = jnp.mean((t - m) ** 2, axis=0, keepdims=True)
    return jax.nn.relu((t - m) / jnp.sqrt(v + 1e-5) * g + b)


def _tc1_body(parts, xv, fc1w, fc1b, g1w, g1b, bn1g, bn1b, fc2w, fc2b,
              prob1, xv3):
    aggr1 = parts[0, :] + parts[1, :]
    xv2 = xv[:][:, None] * fc1w[0, :][None, :] + fc1b[0, :][None, :]
    t = aggr1[:, None] * g1w[0, :][None, :] + g1b[0, :][None, :] + xv2
    prob1[...] = _bn_relu(t, bn1g[...], bn1b[...])
    xv3[...] = _dot(xv2, fc2w[...].T) + fc2b[...]


def _tc_mid_body(parts, xvk, gw, gb, bng, bnb, fcw, fcb, probk, xvk1):
    aggr = parts[0] + parts[1]
    t = _dot(aggr, gw[...].T) + gb[...] + xvk[...]
    probk[...] = _bn_relu(t, bng[...], bnb[...])
    xvk1[...] = _dot(xvk[...], fcw[...].T) + fcb[...]


def _tc4_body(parts, xv5, g4w, g4b, q2w, q2b, q3w, q3b, q1w, q1b, q):
    aggr = parts[0] + parts[1]
    prob = jax.nn.relu(_dot(aggr, g4w[...].T) + g4b[...] + xv5[...])
    gf = jnp.mean(prob, axis=0, keepdims=True)
    wgf = _dot(gf, q2w[...].T) + q2b[...]
    wp = _dot(prob, q3w[...].T) + q3b[...]
    qa = q1w[0, :H]
    qb = q1w[0, H:]
    scal = jnp.sum(jax.nn.relu(wgf)[0, :] * qa)
    q[...] = (_dot(jax.nn.relu(wp), qb[:, None]) + scal) + q1b[0, 0]


def _pc(body, out_shapes, *ins):
    return pl.pallas_call(
        body, out_shape=[jax.ShapeDtypeStruct(s, _F32) for s in out_shapes])(*ins)


def kernel(w, edge_index, batch, x_v, params):
    p = params
    src = edge_index[0]
    dst = edge_index[1]
    srcp = jnp.pad(src.reshape(NW, EW), ((0, 0), (0, EPAD - EW)),
                   constant_values=0).reshape(NW, NCH, C)
    dstp = jnp.pad(dst.reshape(NW, EW), ((0, 0), (0, EPAD - EW)),
                   constant_values=N).reshape(NW, NCH, C)

    def r2(a):
        return a.reshape(1, -1)

    parts1 = _seg1(srcp, dstp, w).reshape(NC, N)
    prob1, xv3 = _pc(
        _tc1_body, [(N, H), (N, H)],
        parts1, x_v,
        r2(p['fc1'][0]), r2(p['fc1'][1]), r2(p['g1'][0]), r2(p['g1'][1]),
        r2(p['bn1'][0]), r2(p['bn1'][1]), p['fc2'][0], r2(p['fc2'][1]))

    parts2 = _segrow(srcp, dstp, prob1)
    prob2, xv4 = _pc(
        _tc_mid_body, [(N, H), (N, H)],
        parts2, xv3, p['g2'][0], r2(p['g2'][1]),
        r2(p['bn2'][0]), r2(p['bn2'][1]), p['fc3'][0], r2(p['fc3'][1]))

    parts3 = _segrow(srcp, dstp, prob2)
    prob3, xv5 = _pc(
        _tc_mid_body, [(N, H), (N, H)],
        parts3, xv4, p['g3'][0], r2(p['g3'][1]),
        r2(p['bn3'][0]), r2(p['bn3'][1]), p['fc4'][0], r2(p['fc4'][1]))

    parts4 = _segrow(srcp, dstp, prob3)
    (q,) = _pc(
        _tc4_body, [(N, 1)],
        parts4, xv5, p['g4'][0], r2(p['g4'][1]),
        p['q2'][0], r2(p['q2'][1]), p['q3'][0], r2(p['q3'][1]),
        p['q1'][0].reshape(1, 2 * H), p['q1'][1].reshape(1, 1))

    Q_dense = q[None, :, :]
    Q_mask = jnp.ones((B, N), dtype=bool)
    adj = jnp.zeros((B, N, N), _F32).at[jnp.zeros_like(src), src, dst].add(1.0)
    return (Q_dense, Q_mask, adj)
